# baseline (speedup 1.0000x reference)
import functools

import jax
import jax.numpy as jnp
from jax.experimental import pallas as pl
from jax.experimental.pallas import tpu as pltpu


def _chamfer_kernel(s_ref, t_ref, fwd_ref, bwd_ref, fmin_scr, bsum_scr):
    j = pl.program_id(1)
    nj = pl.num_programs(1)

    s = s_ref[0]
    t = t_ref[0]

    dot = jax.lax.dot_general(
        s, t, (((1,), (1,)), ((), ())), preferred_element_type=jnp.float32
    )
    s_sq = jnp.sum(s * s, axis=1, keepdims=True)
    t_sq = jnp.sum(t * t, axis=1, keepdims=True).T
    d2 = (s_sq + t_sq) - 2.0 * dot

    tile_fmin = jnp.min(d2, axis=1, keepdims=True)
    tile_bsum = jnp.sum(jnp.min(d2, axis=0))

    @pl.when(j == 0)
    def _():
        fmin_scr[...] = tile_fmin
        bsum_scr[0] = tile_bsum

    @pl.when(j > 0)
    def _():
        fmin_scr[...] = jnp.minimum(fmin_scr[...], tile_fmin)
        bsum_scr[0] = bsum_scr[0] + tile_bsum

    @pl.when(j == nj - 1)
    def _():
        fwd_ref[...] = jnp.full(fwd_ref.shape, jnp.sum(fmin_scr[...]), jnp.float32)
        bwd_ref[...] = jnp.full(bwd_ref.shape, bsum_scr[0], jnp.float32)


@functools.partial(jax.jit, static_argnames=("tm",))
def _chamfer_sums(source_cloud, target_cloud, tm=512):
    B, N, _ = source_cloud.shape
    M = target_cloud.shape[1]
    nj = M // tm

    fwd, bwd = pl.pallas_call(
        _chamfer_kernel,
        grid=(B, nj),
        in_specs=[
            pl.BlockSpec((1, N, 3), lambda b, j: (b, 0, 0)),
            pl.BlockSpec((1, tm, 3), lambda b, j: (b, j, 0)),
        ],
        out_specs=[
            pl.BlockSpec((1, 8, 128), lambda b, j: (b, 0, 0)),
            pl.BlockSpec((1, 8, 128), lambda b, j: (b, 0, 0)),
        ],
        out_shape=[
            jax.ShapeDtypeStruct((B, 8, 128), jnp.float32),
            jax.ShapeDtypeStruct((B, 8, 128), jnp.float32),
        ],
        scratch_shapes=[
            pltpu.VMEM((N, 1), jnp.float32),
            pltpu.SMEM((1,), jnp.float32),
        ],
        compiler_params=pltpu.CompilerParams(
            dimension_semantics=("arbitrary", "arbitrary"),
        ),
    )(source_cloud, target_cloud)
    return fwd[:, 0, 0], bwd[:, 0, 0]


def kernel(source_cloud, target_cloud):
    G = source_cloud.shape[1]
    fwd_sums, bwd_sums = _chamfer_sums(source_cloud, target_cloud)
    return (fwd_sums.mean() + bwd_sums.mean()) / G


# speedup vs baseline: 1.5506x; 1.5506x over previous
import functools

import jax
import jax.numpy as jnp
from jax.experimental import pallas as pl
from jax.experimental.pallas import tpu as pltpu


def _chamfer_kernel(s_ref, t_ref, fwd_ref, bwd_ref, fmin_scr, bsum_scr):
    j = pl.program_id(1)
    nj = pl.num_programs(1)

    s = s_ref[0]
    t = t_ref[0]

    d2 = jax.lax.dot_general(
        s, t, (((1,), (1,)), ((), ())), preferred_element_type=jnp.float32
    )

    tile_fmin = jnp.min(d2, axis=1, keepdims=True)
    tile_bsum = jnp.sum(jnp.min(d2, axis=0))

    @pl.when(j == 0)
    def _():
        fmin_scr[...] = tile_fmin
        bsum_scr[0] = tile_bsum

    @pl.when(j > 0)
    def _():
        fmin_scr[...] = jnp.minimum(fmin_scr[...], tile_fmin)
        bsum_scr[0] = bsum_scr[0] + tile_bsum

    @pl.when(j == nj - 1)
    def _():
        fwd_ref[...] = jnp.full(fwd_ref.shape, jnp.sum(fmin_scr[...]), jnp.float32)
        bwd_ref[...] = jnp.full(bwd_ref.shape, bsum_scr[0], jnp.float32)


@functools.partial(jax.jit, static_argnames=("tm",))
def _chamfer_sums(source_cloud, target_cloud, tm=512):
    B, N, _ = source_cloud.shape
    M = target_cloud.shape[1]
    nj = M // tm

    s = source_cloud[:, :, :3]
    t = target_cloud[:, :, :3]
    s_sq = jnp.sum(s * s, axis=2, keepdims=True)
    t_sq = jnp.sum(t * t, axis=2, keepdims=True)
    ones_s = jnp.ones((B, N, 1), jnp.float32)
    ones_t = jnp.ones((B, M, 1), jnp.float32)
    zeros_s = jnp.zeros((B, N, 3), jnp.float32)
    zeros_t = jnp.zeros((B, M, 3), jnp.float32)
    s_aug = jnp.concatenate([-2.0 * s, s_sq, ones_s, zeros_s], axis=2)
    t_aug = jnp.concatenate([t, ones_t, t_sq, zeros_t], axis=2)

    fwd, bwd = pl.pallas_call(
        _chamfer_kernel,
        grid=(B, nj),
        in_specs=[
            pl.BlockSpec((1, N, 8), lambda b, j: (b, 0, 0)),
            pl.BlockSpec((1, tm, 8), lambda b, j: (b, j, 0)),
        ],
        out_specs=[
            pl.BlockSpec((1, 8, 128), lambda b, j: (b, 0, 0)),
            pl.BlockSpec((1, 8, 128), lambda b, j: (b, 0, 0)),
        ],
        out_shape=[
            jax.ShapeDtypeStruct((B, 8, 128), jnp.float32),
            jax.ShapeDtypeStruct((B, 8, 128), jnp.float32),
        ],
        scratch_shapes=[
            pltpu.VMEM((N, 1), jnp.float32),
            pltpu.SMEM((1,), jnp.float32),
        ],
        compiler_params=pltpu.CompilerParams(
            dimension_semantics=("parallel", "arbitrary"),
        ),
    )(s_aug, t_aug)
    return fwd[:, 0, 0], bwd[:, 0, 0]


def kernel(source_cloud, target_cloud):
    G = source_cloud.shape[1]
    fwd_sums, bwd_sums = _chamfer_sums(source_cloud, target_cloud)
    return (fwd_sums.mean() + bwd_sums.mean()) / G
